# baseline (speedup 1.0000x reference)
import functools

import jax
import jax.numpy as jnp
import numpy as np
from jax.experimental import pallas as pl
from jax.experimental.pallas import tpu as pltpu

N_HEADS_ = 16
N_KV_ = 4
G_ = N_HEADS_ // N_KV_
D_ = 128
SEQ_ = 2048
BS_ = 64
NC_ = SEQ_ // BS_
KSEL_ = 16
WIN_ = 512
TQ_ = 256
TK_ = 256
SCALE_ = D_ ** -0.5
NEG_ = -1e9
C2_ = SCALE_ * 1.4426950408889634


def _proj_kernel(x_ref, wq_ref, wk_ref, wv_ref, wg_ref, cos_ref, sin_ref,
                 qo_ref, ko_ref, vo_ref, go_ref, kc_ref, vc_ref):
    x = x_ref[...]
    f32 = jnp.float32
    q = jax.lax.dot(x, wq_ref[...], preferred_element_type=f32)
    k = jax.lax.dot(x, wk_ref[...], preferred_element_type=f32)
    v = jax.lax.dot(x, wv_ref[...], preferred_element_type=f32)
    g = jax.lax.dot(x, wg_ref[...], preferred_element_type=f32)
    go_ref[...] = jax.nn.sigmoid(g)
    c = cos_ref[...]
    s = sin_ref[...]
    def rope(h):
        a = h[:, :64]
        b = h[:, 64:]
        return jnp.concatenate([a * c - b * s, a * s + b * c], axis=1)
    for h in range(N_HEADS_):
        qo_ref[:, h * D_:(h + 1) * D_] = rope(
            q[:, h * D_:(h + 1) * D_]).astype(jnp.bfloat16)
    kr = jnp.concatenate(
        [rope(k[:, h * D_:(h + 1) * D_]) for h in range(N_KV_)], axis=1)
    ko_ref[...] = kr.astype(jnp.bfloat16)
    for h in range(N_KV_):
        vo_ref[h, :, :D_] = v[:, h * D_:(h + 1) * D_].astype(jnp.bfloat16)
        vo_ref[h, :, D_:] = jnp.ones((TQ_, 8), jnp.bfloat16)
    kc_ref[0] = jnp.mean(kr.reshape(TQ_ // BS_, BS_, N_KV_ * D_),
                         axis=1).astype(jnp.bfloat16)
    vc_ref[0] = jnp.mean(v.reshape(TQ_ // BS_, BS_, N_KV_ * D_),
                         axis=1).astype(jnp.bfloat16)


def _attn_kernel(q_ref, k_ref, v_ref, kc_ref, vc_ref, g_ref, o_ref,
                 ps_ref, pw_ref):
    qb = pl.program_id(1)
    f32 = jnp.float32
    bf16 = jnp.bfloat16
    q = q_ref[...]
    kc = kc_ref[...]
    vc = vc_ref[...]
    qs = [q[:, g * D_:(g + 1) * D_] for g in range(G_)]
    q4 = jnp.concatenate(qs, axis=0)

    pos_l = qb * TQ_ + jax.lax.broadcasted_iota(jnp.int32, (1, TQ_), 1)
    iota_b = jax.lax.broadcasted_iota(jnp.int32, (NC_, TQ_), 0)
    cmp_vis = ((iota_b + 1) * BS_ - 1) <= pos_l
    pos_s = qb * TQ_ + jax.lax.broadcasted_iota(jnp.int32, (TQ_, 1), 0)
    valid_cmp = (pos_s >= BS_ - 1).astype(f32)

    scores = jnp.zeros((NC_, TQ_), f32)
    o_cmps = []
    for g in range(G_):
        lg = jax.lax.dot_general(kc, qs[g], (((1,), (1,)), ((), ())),
                                 preferred_element_type=f32) * SCALE_
        lg = jnp.where(cmp_vis, lg, NEG_)
        m = jnp.max(lg, axis=0, keepdims=True)
        e = jnp.exp(lg - m)
        p = e * (1.0 / jnp.sum(e, axis=0, keepdims=True))
        o_cmps.append(jax.lax.dot_general(
            p.astype(bf16), vc, (((0,), (0,)), ((), ())),
            preferred_element_type=f32) * valid_cmp)
        scores = scores + p

    force = ((iota_b == pos_l // BS_).astype(f32)
             + (iota_b == 0).astype(f32))
    sel_vis = (iota_b * BS_) <= pos_l
    s = jnp.where(sel_vis, scores + 1e4 * force, NEG_)
    cnt = jnp.zeros((NC_, TQ_), f32)
    for i in range(NC_):
        cnt = cnt + (s[i:i + 1, :] > s).astype(f32)
    selT = (cnt < KSEL_).astype(bf16)
    blk_i = jax.lax.broadcasted_iota(jnp.int32, (NC_, 2), 0)
    half_i = jax.lax.broadcasted_iota(jnp.int32, (NC_, 2), 1)
    pow2 = jnp.where(blk_i // 16 == half_i,
                     jnp.exp2((blk_i - 16 * half_i).astype(f32)),
                     0.0).astype(bf16)
    bits = jax.lax.dot(selT.T, pow2, preferred_element_type=f32)
    bits4 = jnp.concatenate([bits] * G_, axis=0)
    bits_lo = bits4[:, 0:1]
    bits_hi = bits4[:, 1:2]
    lane_blk = jax.lax.broadcasted_iota(jnp.int32, (1, TK_), 1) // BS_
    invp_lane = jnp.exp2(-lane_blk.astype(f32))

    rown = jax.lax.broadcasted_iota(jnp.int32, (G_ * TQ_, TK_), 0) % TQ_
    iota_t = jax.lax.broadcasted_iota(jnp.int32, (G_ * TQ_, TK_), 1)
    diff = iota_t - rown

    n_tiles = SEQ_ // TK_
    zeros_t = jnp.zeros((G_ * TQ_, TK_), bf16)
    def tile_logits(j):
        k_t = k_ref[j * TK_:(j + 1) * TK_, :]
        return jax.lax.dot_general(q4, k_t, (((1,), (1,)), ((), ())),
                                   preferred_element_type=f32)

    def tile_selm(j):
        bcol = bits_lo if j < 16 // (TK_ // BS_) else bits_hi
        x = bcol * (invp_lane * (2.0 ** -((TK_ // BS_) * (j % 4))))
        t = jnp.floor(x)
        return (t - 2.0 * jnp.floor(t * 0.5)) > 0.5

    for j in range(n_tiles):
        @pl.when(qb == j)
        def _diag(j=j):
            pc = jnp.exp2(jnp.where(diff <= 0, tile_logits(j), NEG_)
                          * C2_).astype(bf16)
            ps_ref[:, j * TK_:(j + 1) * TK_] = jnp.where(
                tile_selm(j), pc, jnp.bfloat16(0.0))
            pw_ref[:, j * TK_:(j + 1) * TK_] = pc

        if j + 1 < n_tiles:
            @pl.when(qb == j + 1)
            def _win1(j=j):
                e = jnp.exp2(tile_logits(j) * C2_).astype(bf16)
                ps_ref[:, j * TK_:(j + 1) * TK_] = jnp.where(
                    tile_selm(j), e, jnp.bfloat16(0.0))
                pw_ref[:, j * TK_:(j + 1) * TK_] = e

        if j + 2 < n_tiles:
            @pl.when(qb == j + 2)
            def _win2(j=j):
                e = jnp.exp2(tile_logits(j) * C2_).astype(bf16)
                ps_ref[:, j * TK_:(j + 1) * TK_] = jnp.where(
                    tile_selm(j), e, jnp.bfloat16(0.0))
                pw_ref[:, j * TK_:(j + 1) * TK_] = jnp.where(
                    diff > 0, e, jnp.bfloat16(0.0))

        if j + 3 < n_tiles:
            @pl.when(qb >= j + 3)
            def _inner(j=j):
                ps_ref[:, j * TK_:(j + 1) * TK_] = jnp.exp2(
                    jnp.where(tile_selm(j), tile_logits(j), NEG_)
                    * C2_).astype(bf16)

            @pl.when(qb == j + 3)
            def _pwz(j=j):
                pw_ref[:, j * TK_:(j + 1) * TK_] = zeros_t

        if j % 4 != 0:
            @pl.when(qb == (0 if j < 4 else 4))
            def _zero_tail(j=j):
                ps_ref[:, j * TK_:(j + 1) * TK_] = zeros_t
                pw_ref[:, j * TK_:(j + 1) * TK_] = zeros_t

    va0 = v_ref[0, :4 * TK_, :]
    d_s0 = jax.lax.dot(ps_ref[:, :4 * TK_], va0, preferred_element_type=f32)

    def with_seg1():
        va1 = v_ref[0, 4 * TK_:, :]
        return d_s0 + jax.lax.dot(ps_ref[:, 4 * TK_:], va1,
                                  preferred_element_type=f32)
    acc_s = jax.lax.cond(qb >= 4, with_seg1, lambda: d_s0)

    def w_lo():
        return jax.lax.dot(pw_ref[:, :4 * TK_], va0,
                           preferred_element_type=f32)

    def w_hi():
        va1 = v_ref[0, 4 * TK_:, :]
        return jax.lax.dot(pw_ref[:, 4 * TK_:], va1,
                           preferred_element_type=f32)
    acc_w = jax.lax.cond(
        qb <= 3, w_lo,
        lambda: jax.lax.cond(qb <= 5, lambda: w_lo() + w_hi(), w_hi))
    a_s = acc_s[:, :D_]
    a_w = acc_w[:, :D_]
    inv_s = 1.0 / acc_s[:, D_:D_ + 1]
    inv_w = 1.0 / acc_w[:, D_:D_ + 1]

    gt = g_ref[0]
    for g in range(G_):
        r0 = g * TQ_
        o_slc = a_s[r0:r0 + TQ_] * inv_s[r0:r0 + TQ_]
        o_swa = a_w[r0:r0 + TQ_] * inv_w[r0:r0 + TQ_]
        out = (gt[:, g:g + 1] * o_cmps[g]
               + gt[:, G_ + g:G_ + g + 1] * o_slc
               + gt[:, 2 * G_ + g:2 * G_ + g + 1] * o_swa)
        o_ref[:, g * D_:(g + 1) * D_] = out


def _out_kernel(x_ref, w_ref, o_ref):
    o_ref[...] = jax.lax.dot(x_ref[...].astype(jnp.bfloat16), w_ref[...],
                             preferred_element_type=jnp.float32)


@functools.partial(jax.jit, static_argnums=())
def kernel(x, start_pos, freqs_cis, mask, wq, wk, wv, wg, wo):
    del start_pos, mask
    S, DIM = SEQ_, N_HEADS_ * D_
    xb = x.reshape(S, DIM).astype(jnp.bfloat16)

    perm = np.arange(D_).reshape(D_ // 2, 2).T.reshape(-1)
    qperm = np.concatenate([perm + h * D_ for h in range(N_HEADS_)])
    kperm = np.concatenate([perm + h * D_ for h in range(N_KV_)])
    wq_p = wq[:, qperm].astype(jnp.bfloat16)
    wk_p = wk[:, kperm].astype(jnp.bfloat16)
    wv_b = wv.astype(jnp.bfloat16)
    gperm = np.asarray([3 * (4 * hk + g) + j for hk in range(N_KV_)
                        for j in range(3) for g in range(G_)])
    wg_p = wg[:, gperm].astype(jnp.bfloat16)
    cos = freqs_cis[:, :, 0]
    sin = freqs_cis[:, :, 1]

    n_row = S // TQ_
    f32 = jnp.float32
    bf16 = jnp.bfloat16
    row_spec = lambda w: pl.BlockSpec((TQ_, w), lambda i: (i, 0))
    pin_spec = lambda a: pl.BlockSpec(a.shape, lambda i: (0, 0))
    q_r, k_r, v_r, gates, k_cmp, v_cmp = pl.pallas_call(
        _proj_kernel,
        grid=(n_row,),
        in_specs=[row_spec(DIM), pin_spec(wq_p), pin_spec(wk_p),
                  pin_spec(wv_b), pin_spec(wg_p), row_spec(64), row_spec(64)],
        out_specs=[row_spec(DIM), row_spec(N_KV_ * D_),
                   pl.BlockSpec((N_KV_, TQ_, D_ + 8), lambda i: (0, i, 0)),
                   row_spec(3 * N_HEADS_),
                   pl.BlockSpec((1, TQ_ // BS_, N_KV_ * D_),
                                lambda i: (i, 0, 0)),
                   pl.BlockSpec((1, TQ_ // BS_, N_KV_ * D_),
                                lambda i: (i, 0, 0))],
        out_shape=[jax.ShapeDtypeStruct((S, DIM), bf16),
                   jax.ShapeDtypeStruct((S, N_KV_ * D_), bf16),
                   jax.ShapeDtypeStruct((N_KV_, S, D_ + 8), bf16),
                   jax.ShapeDtypeStruct((S, 3 * N_HEADS_), f32),
                   jax.ShapeDtypeStruct((n_row, TQ_ // BS_, N_KV_ * D_), bf16),
                   jax.ShapeDtypeStruct((n_row, TQ_ // BS_, N_KV_ * D_), bf16)],
    )(xb, wq_p, wk_p, wv_b, wg_p, cos, sin)
    k_cmp = k_cmp.reshape(NC_, N_KV_ * D_)
    v_cmp = v_cmp.reshape(NC_, N_KV_ * D_)

    gates_r = gates.reshape(S, N_KV_, 3 * G_).transpose(1, 0, 2)

    o = pl.pallas_call(
        _attn_kernel,
        grid=(N_KV_, n_row),
        in_specs=[
            pl.BlockSpec((TQ_, G_ * D_), lambda h, qb: (qb, h)),
            pl.BlockSpec((S, D_), lambda h, qb: (0, h)),
            pl.BlockSpec((1, S, D_ + 8), lambda h, qb: (h, 0, 0)),
            pl.BlockSpec((NC_, D_), lambda h, qb: (0, h)),
            pl.BlockSpec((NC_, D_), lambda h, qb: (0, h)),
            pl.BlockSpec((1, TQ_, 3 * G_), lambda h, qb: (h, qb, 0)),
        ],
        out_specs=pl.BlockSpec((TQ_, G_ * D_), lambda h, qb: (qb, h)),
        out_shape=jax.ShapeDtypeStruct((S, DIM), f32),
        scratch_shapes=[pltpu.VMEM((G_ * TQ_, SEQ_), bf16),
                        pltpu.VMEM((G_ * TQ_, SEQ_), bf16)],
    )(q_r, k_r, v_r, k_cmp, v_cmp, gates_r)

    out = pl.pallas_call(
        _out_kernel,
        grid=(n_row,),
        in_specs=[row_spec(DIM), pin_spec(wo)],
        out_specs=row_spec(DIM),
        out_shape=jax.ShapeDtypeStruct((S, DIM), f32),
    )(o, wo.astype(jnp.bfloat16))
    return out.reshape(1, S, DIM)


# speedup vs baseline: 1.0155x; 1.0155x over previous
import functools

import jax
import jax.numpy as jnp
import numpy as np
from jax.experimental import pallas as pl
from jax.experimental.pallas import tpu as pltpu

N_HEADS_ = 16
N_KV_ = 4
G_ = N_HEADS_ // N_KV_
D_ = 128
SEQ_ = 2048
BS_ = 64
NC_ = SEQ_ // BS_
KSEL_ = 16
WIN_ = 512
TQ_ = 256
TK_ = 256
SCALE_ = D_ ** -0.5
NEG_ = -1e9
C2_ = SCALE_ * 1.4426950408889634


def _proj_kernel(x_ref, wq_ref, wk_ref, wv_ref, wg_ref, cos_ref, sin_ref,
                 qo_ref, ko_ref, vo_ref, go_ref, kc_ref, vc_ref):
    x = x_ref[...]
    f32 = jnp.float32
    q = jax.lax.dot(x, wq_ref[...], preferred_element_type=f32)
    k = jax.lax.dot(x, wk_ref[...], preferred_element_type=f32)
    v = jax.lax.dot(x, wv_ref[...], preferred_element_type=f32)
    g = jax.lax.dot(x, wg_ref[...], preferred_element_type=f32)
    go_ref[...] = jax.nn.sigmoid(g)
    c = cos_ref[...]
    s = sin_ref[...]
    def rope(h):
        a = h[:, :64]
        b = h[:, 64:]
        return jnp.concatenate([a * c - b * s, a * s + b * c], axis=1)
    for h in range(N_HEADS_):
        qo_ref[:, h * D_:(h + 1) * D_] = rope(
            q[:, h * D_:(h + 1) * D_]).astype(jnp.bfloat16)
    kr = jnp.concatenate(
        [rope(k[:, h * D_:(h + 1) * D_]) for h in range(N_KV_)], axis=1)
    ko_ref[...] = kr.astype(jnp.bfloat16)
    for h in range(N_KV_):
        vo_ref[h, :, :D_] = v[:, h * D_:(h + 1) * D_].astype(jnp.bfloat16)
        vo_ref[h, :, D_:] = jnp.ones((TQ_, 8), jnp.bfloat16)
    kc_ref[0] = jnp.mean(kr.reshape(TQ_ // BS_, BS_, N_KV_ * D_),
                         axis=1).astype(jnp.bfloat16)
    vc_ref[0] = jnp.mean(v.reshape(TQ_ // BS_, BS_, N_KV_ * D_),
                         axis=1).astype(jnp.bfloat16)


def _attn_kernel(q_ref, k_ref, v_ref, kc_ref, vc_ref, g_ref, o_ref,
                 ps_ref, pw_ref):
    qb = pl.program_id(1)
    f32 = jnp.float32
    bf16 = jnp.bfloat16
    q = q_ref[...]
    kc = kc_ref[...]
    vc = vc_ref[...]
    qs = [q[:, g * D_:(g + 1) * D_] for g in range(G_)]
    q4 = jnp.concatenate(qs, axis=0)

    pos_l = qb * TQ_ + jax.lax.broadcasted_iota(jnp.int32, (1, TQ_), 1)
    iota_b = jax.lax.broadcasted_iota(jnp.int32, (NC_, TQ_), 0)
    cmp_vis = ((iota_b + 1) * BS_ - 1) <= pos_l
    pos_s = qb * TQ_ + jax.lax.broadcasted_iota(jnp.int32, (TQ_, 1), 0)
    valid_cmp = (pos_s >= BS_ - 1).astype(f32)

    scores = jnp.zeros((NC_, TQ_), f32)
    o_cmps = []
    for g in range(G_):
        lg = jax.lax.dot_general(kc, qs[g], (((1,), (1,)), ((), ())),
                                 preferred_element_type=f32) * SCALE_
        lg = jnp.where(cmp_vis, lg, NEG_)
        m = jnp.max(lg, axis=0, keepdims=True)
        e = jnp.exp(lg - m)
        p = e * (1.0 / jnp.sum(e, axis=0, keepdims=True))
        o_cmps.append(jax.lax.dot_general(
            p.astype(bf16), vc, (((0,), (0,)), ((), ())),
            preferred_element_type=f32) * valid_cmp)
        scores = scores + p

    force = ((iota_b == pos_l // BS_).astype(f32)
             + (iota_b == 0).astype(f32))
    sel_vis = (iota_b * BS_) <= pos_l
    s = jnp.where(sel_vis, scores + 1e4 * force, NEG_)
    cnt = jnp.zeros((NC_, TQ_), f32)
    for i in range(NC_):
        cnt = cnt + (s[i:i + 1, :] > s).astype(f32)
    selT = (cnt < KSEL_).astype(bf16)
    blk_i = jax.lax.broadcasted_iota(jnp.int32, (NC_, 2), 0)
    half_i = jax.lax.broadcasted_iota(jnp.int32, (NC_, 2), 1)
    pow2 = jnp.where(blk_i // 16 == half_i,
                     jnp.exp2((blk_i - 16 * half_i).astype(f32)),
                     0.0).astype(bf16)
    bits = jax.lax.dot(selT.T, pow2, preferred_element_type=f32)
    bits4 = jnp.concatenate([bits] * G_, axis=0)
    bits_lo = bits4[:, 0:1]
    bits_hi = bits4[:, 1:2]
    lane_blk = jax.lax.broadcasted_iota(jnp.int32, (1, TK_), 1) // BS_
    invp_lane = jnp.exp2(-lane_blk.astype(f32))

    rown = jax.lax.broadcasted_iota(jnp.int32, (G_ * TQ_, TK_), 0) % TQ_
    iota_t = jax.lax.broadcasted_iota(jnp.int32, (G_ * TQ_, TK_), 1)
    diff = iota_t - rown

    n_tiles = SEQ_ // TK_
    zeros_t = jnp.zeros((G_ * TQ_, TK_), bf16)
    def tile_logits(j):
        k_t = k_ref[j * TK_:(j + 1) * TK_, :]
        return jax.lax.dot_general(q4, k_t, (((1,), (1,)), ((), ())),
                                   preferred_element_type=f32)

    def tile_selm(j):
        bcol = bits_lo if j < 16 // (TK_ // BS_) else bits_hi
        x = bcol * (invp_lane * (2.0 ** -((TK_ // BS_) * (j % 4))))
        t = jnp.floor(x)
        return (t - 2.0 * jnp.floor(t * 0.5)) > 0.5

    for j in range(n_tiles):
        @pl.when(qb == j)
        def _diag(j=j):
            pc = jnp.exp2(jnp.where(diff <= 0, tile_logits(j), NEG_)
                          * C2_).astype(bf16)
            ps_ref[:, j * TK_:(j + 1) * TK_] = jnp.where(
                tile_selm(j), pc, jnp.bfloat16(0.0))
            pw_ref[:, j * TK_:(j + 1) * TK_] = pc

        if j + 1 < n_tiles:
            @pl.when(qb == j + 1)
            def _win1(j=j):
                e = jnp.exp2(tile_logits(j) * C2_).astype(bf16)
                ps_ref[:, j * TK_:(j + 1) * TK_] = jnp.where(
                    tile_selm(j), e, jnp.bfloat16(0.0))
                pw_ref[:, j * TK_:(j + 1) * TK_] = e

        if j + 2 < n_tiles:
            @pl.when(qb == j + 2)
            def _win2(j=j):
                e = jnp.exp2(tile_logits(j) * C2_).astype(bf16)
                ps_ref[:, j * TK_:(j + 1) * TK_] = jnp.where(
                    tile_selm(j), e, jnp.bfloat16(0.0))
                pw_ref[:, j * TK_:(j + 1) * TK_] = jnp.where(
                    diff > 0, e, jnp.bfloat16(0.0))

        if j + 3 < n_tiles:
            @pl.when(qb >= j + 3)
            def _inner(j=j):
                ps_ref[:, j * TK_:(j + 1) * TK_] = jnp.exp2(
                    jnp.where(tile_selm(j), tile_logits(j), NEG_)
                    * C2_).astype(bf16)

            @pl.when(qb == j + 3)
            def _pwz(j=j):
                pw_ref[:, j * TK_:(j + 1) * TK_] = zeros_t

        if j % 4 != 0:
            @pl.when(qb == (0 if j < 4 else 4))
            def _zero_tail(j=j):
                ps_ref[:, j * TK_:(j + 1) * TK_] = zeros_t
                pw_ref[:, j * TK_:(j + 1) * TK_] = zeros_t

    va0 = v_ref[0, :4 * TK_, :]
    d_s0 = jax.lax.dot(ps_ref[:, :4 * TK_], va0, preferred_element_type=f32)
    d_w0 = jax.lax.dot(pw_ref[:, :4 * TK_], va0, preferred_element_type=f32)

    def with_seg1():
        va1 = v_ref[0, 4 * TK_:, :]
        return (d_s0 + jax.lax.dot(ps_ref[:, 4 * TK_:], va1,
                                   preferred_element_type=f32),
                d_w0 + jax.lax.dot(pw_ref[:, 4 * TK_:], va1,
                                   preferred_element_type=f32))
    acc_s, acc_w = jax.lax.cond(qb >= 4, with_seg1, lambda: (d_s0, d_w0))
    a_s = acc_s[:, :D_]
    a_w = acc_w[:, :D_]
    inv_s = 1.0 / acc_s[:, D_:D_ + 1]
    inv_w = 1.0 / acc_w[:, D_:D_ + 1]

    gt = g_ref[0]
    for g in range(G_):
        r0 = g * TQ_
        o_slc = a_s[r0:r0 + TQ_] * inv_s[r0:r0 + TQ_]
        o_swa = a_w[r0:r0 + TQ_] * inv_w[r0:r0 + TQ_]
        out = (gt[:, g:g + 1] * o_cmps[g]
               + gt[:, G_ + g:G_ + g + 1] * o_slc
               + gt[:, 2 * G_ + g:2 * G_ + g + 1] * o_swa)
        o_ref[:, g * D_:(g + 1) * D_] = out


def _out_kernel(x_ref, w_ref, o_ref):
    o_ref[...] = jax.lax.dot(x_ref[...].astype(jnp.bfloat16), w_ref[...],
                             preferred_element_type=jnp.float32)


@functools.partial(jax.jit, static_argnums=())
def kernel(x, start_pos, freqs_cis, mask, wq, wk, wv, wg, wo):
    del start_pos, mask
    S, DIM = SEQ_, N_HEADS_ * D_
    xb = x.reshape(S, DIM).astype(jnp.bfloat16)

    perm = np.arange(D_).reshape(D_ // 2, 2).T.reshape(-1)
    qperm = np.concatenate([perm + h * D_ for h in range(N_HEADS_)])
    kperm = np.concatenate([perm + h * D_ for h in range(N_KV_)])
    wq_p = wq[:, qperm].astype(jnp.bfloat16)
    wk_p = wk[:, kperm].astype(jnp.bfloat16)
    wv_b = wv.astype(jnp.bfloat16)
    gperm = np.asarray([3 * (4 * hk + g) + j for hk in range(N_KV_)
                        for j in range(3) for g in range(G_)])
    wg_p = wg[:, gperm].astype(jnp.bfloat16)
    cos = freqs_cis[:, :, 0]
    sin = freqs_cis[:, :, 1]

    n_row = S // TQ_
    f32 = jnp.float32
    bf16 = jnp.bfloat16
    row_spec = lambda w: pl.BlockSpec((TQ_, w), lambda i: (i, 0))
    pin_spec = lambda a: pl.BlockSpec(a.shape, lambda i: (0, 0))
    q_r, k_r, v_r, gates, k_cmp, v_cmp = pl.pallas_call(
        _proj_kernel,
        grid=(n_row,),
        in_specs=[row_spec(DIM), pin_spec(wq_p), pin_spec(wk_p),
                  pin_spec(wv_b), pin_spec(wg_p), row_spec(64), row_spec(64)],
        out_specs=[row_spec(DIM), row_spec(N_KV_ * D_),
                   pl.BlockSpec((N_KV_, TQ_, D_ + 8), lambda i: (0, i, 0)),
                   row_spec(3 * N_HEADS_),
                   pl.BlockSpec((1, TQ_ // BS_, N_KV_ * D_),
                                lambda i: (i, 0, 0)),
                   pl.BlockSpec((1, TQ_ // BS_, N_KV_ * D_),
                                lambda i: (i, 0, 0))],
        out_shape=[jax.ShapeDtypeStruct((S, DIM), bf16),
                   jax.ShapeDtypeStruct((S, N_KV_ * D_), bf16),
                   jax.ShapeDtypeStruct((N_KV_, S, D_ + 8), bf16),
                   jax.ShapeDtypeStruct((S, 3 * N_HEADS_), f32),
                   jax.ShapeDtypeStruct((n_row, TQ_ // BS_, N_KV_ * D_), bf16),
                   jax.ShapeDtypeStruct((n_row, TQ_ // BS_, N_KV_ * D_), bf16)],
    )(xb, wq_p, wk_p, wv_b, wg_p, cos, sin)
    k_cmp = k_cmp.reshape(NC_, N_KV_ * D_)
    v_cmp = v_cmp.reshape(NC_, N_KV_ * D_)

    gates_r = gates.reshape(S, N_KV_, 3 * G_).transpose(1, 0, 2)

    o = pl.pallas_call(
        _attn_kernel,
        grid=(N_KV_, n_row),
        in_specs=[
            pl.BlockSpec((TQ_, G_ * D_), lambda h, qb: (qb, h)),
            pl.BlockSpec((S, D_), lambda h, qb: (0, h)),
            pl.BlockSpec((1, S, D_ + 8), lambda h, qb: (h, 0, 0)),
            pl.BlockSpec((NC_, D_), lambda h, qb: (0, h)),
            pl.BlockSpec((NC_, D_), lambda h, qb: (0, h)),
            pl.BlockSpec((1, TQ_, 3 * G_), lambda h, qb: (h, qb, 0)),
        ],
        out_specs=pl.BlockSpec((TQ_, G_ * D_), lambda h, qb: (qb, h)),
        out_shape=jax.ShapeDtypeStruct((S, DIM), f32),
        scratch_shapes=[pltpu.VMEM((G_ * TQ_, SEQ_), bf16),
                        pltpu.VMEM((G_ * TQ_, SEQ_), bf16)],
    )(q_r, k_r, v_r, k_cmp, v_cmp, gates_r)

    out = pl.pallas_call(
        _out_kernel,
        grid=(n_row,),
        in_specs=[row_spec(DIM), pin_spec(wo)],
        out_specs=row_spec(DIM),
        out_shape=jax.ShapeDtypeStruct((S, DIM), f32),
    )(o, wo.astype(jnp.bfloat16))
    return out.reshape(1, S, DIM)
